# baseline (speedup 1.0000x reference)
import functools

import jax
import jax.numpy as jnp
from jax import lax
from jax.experimental import pallas as pl
from jax.experimental.pallas import tpu as pltpu
from jax.experimental.pallas import tpu_sc as plsc

NITER = 20000
NLOADS = 64


def _micro_kernel(out_hbm, buf_v, acc_v, stage_v, sem):
    lanes = lax.iota(jnp.int32, 16)
    col_ids = [jnp.full((16,), k, jnp.int32) for k in range(16)]

    def body(i, carry):
        vs = [plsc.load_gather(stage_v, [lanes, col_ids[k]]) for k in range(16)]
        while len(vs) > 1:
            vs = [vs[i2] + vs[i2 + 1] for i2 in range(0, len(vs), 2)]
        acc_v[pl.ds(0, 16)] = vs[0]
        return carry

    lax.fori_loop(0, NITER, body, 0, unroll=False)
    pltpu.sync_copy(acc_v, out_hbm.at[lax.axis_index("s") * 2 + lax.axis_index("c")])


def kernel(input, w2v, nsi):
    mesh = plsc.VectorSubcoreMesh(core_axis_name="c", subcore_axis_name="s")
    kern = functools.partial(
        pl.kernel,
        mesh=mesh,
        out_type=jax.ShapeDtypeStruct((32, 16), jnp.float32),
        scratch_types=[
            pltpu.VMEM((NLOADS * 16,), jnp.float32),
            pltpu.VMEM((16,), jnp.float32),
            pltpu.VMEM((16, 17), jnp.float32),
            pltpu.SemaphoreType.DMA,
        ],
        compiler_params=pltpu.CompilerParams(needs_layout_passes=False),
    )(_micro_kernel)
    out = kern()
    return jnp.sum(out)


# speedup vs baseline: 30.0014x; 30.0014x over previous
import functools

import jax
import jax.numpy as jnp
from jax import lax
from jax.experimental import pallas as pl
from jax.experimental.pallas import tpu as pltpu
from jax.experimental.pallas import tpu_sc as plsc

WINDOW = 5
TID = 2
NS = 8
NDIM = 128
BATCH = 4096
NSC = 36
NWORKERS = 32
B_PER_W = BATCH // NWORKERS
CHUNK_B = 8
NCHUNKS = BATCH // CHUNK_B
CHUNKS_PER_W = NCHUNKS // NWORKERS
ROWS_PER_CHUNK = CHUNK_B * NSC
GATHER_SPLIT = 3
ROWS_PER_GATHER = ROWS_PER_CHUNK // GATHER_SPLIT
SCORES_PER_W = B_PER_W * NSC
SC_BUF_PAD = SCORES_PER_W + NDIM
NLANE = 16
NBUF = 2


def _sc_scores_kernel(w2v_hbm, idx3_hbm, cidx_hbm, out_hbm,
                      idx_v, cidx_v, vi_v, rows0_v, rows1_v,
                      sc_v, sem_vi, sem0, sem1):
    wid = lax.axis_index("s") * 2 + lax.axis_index("c")
    rows_bufs = (rows0_v, rows1_v)
    sems = (sem0, sem1)

    pltpu.sync_copy(idx3_hbm.at[pl.ds(wid * CHUNKS_PER_W, CHUNKS_PER_W)], idx_v)
    pltpu.sync_copy(cidx_hbm.at[wid], cidx_v)
    pltpu.async_copy(w2v_hbm.at[cidx_v], vi_v, sem_vi).wait()

    def issue(c, buf, sem):
        for k in range(GATHER_SPLIT):
            pltpu.async_copy(
                w2v_hbm.at[idx_v.at[c].at[k]],
                rows_bufs[buf].at[pl.ds(k * ROWS_PER_GATHER, ROWS_PER_GATHER)],
                sem,
            )

    def drain(buf, sem):
        pltpu.make_async_copy(
            w2v_hbm.at[pl.ds(0, ROWS_PER_CHUNK)], rows_bufs[buf], sem
        ).wait()

    lane0 = lax.iota(jnp.int32, NLANE) == 0

    def _tree_sum(vs):
        while len(vs) > 1:
            vs = [vs[i] + vs[i + 1] for i in range(0, len(vs) - 1, 2)] + (
                [vs[-1]] if len(vs) % 2 else [])
        return vs[0]

    def compute(c, buf):
        rows_v = rows_bufs[buf]

        def store_score(pos, s):
            plsc.store_compressed(
                sc_v.at[pl.ds(pos, NLANE)],
                jnp.full((NLANE,), s, jnp.float32), mask=lane0)

        def b_body(b, carry2):
            row = c * CHUNK_B + b
            vi_regs = [vi_v[row, pl.ds(k * NLANE, NLANE)]
                       for k in range(NDIM // NLANE)]
            base = row * NSC

            def load_rows(j):
                r = b * NSC + j
                return [rows_v[r, pl.ds(k * NLANE, NLANE)]
                        for k in range(NDIM // NLANE)]

            regs = load_rows(0)
            prev_s = None
            for j in range(NSC):
                prods = [regs[k] * vi_regs[k] for k in range(NDIM // NLANE)]
                if j + 1 < NSC:
                    regs = load_rows(j + 1)
                s = jnp.sum(_tree_sum(prods))
                if prev_s is not None:
                    store_score(base + j - 1, prev_s)
                prev_s = s
            store_score(base + NSC - 1, prev_s)
            return carry2

        lax.fori_loop(0, CHUNK_B, b_body, 0, unroll=False)

    for buf in range(NBUF):
        issue(buf, buf, sems[buf])

    def ring_body(it, carry):
        c0 = it * NBUF
        for off in range(NBUF):
            c = c0 + off
            drain(off, sems[off])
            compute(c, off)

            @pl.when(c + NBUF < CHUNKS_PER_W)
            def _():
                issue(c + NBUF, off, sems[off])
        return carry

    lax.fori_loop(0, CHUNKS_PER_W // NBUF, ring_body, 0, unroll=False)
    pltpu.sync_copy(sc_v, out_hbm.at[wid])


def _sc_scores(w2v, idx3, cidx):
    mesh = plsc.VectorSubcoreMesh(core_axis_name="c", subcore_axis_name="s")
    kern = functools.partial(
        pl.kernel,
        mesh=mesh,
        out_type=jax.ShapeDtypeStruct((NWORKERS, SC_BUF_PAD), jnp.float32),
        scratch_types=[
            pltpu.VMEM((CHUNKS_PER_W, GATHER_SPLIT, ROWS_PER_GATHER), jnp.int32),
            pltpu.VMEM((B_PER_W,), jnp.int32),
            pltpu.VMEM((B_PER_W, NDIM), jnp.float32),
            pltpu.VMEM((ROWS_PER_CHUNK, NDIM), jnp.float32),
            pltpu.VMEM((ROWS_PER_CHUNK, NDIM), jnp.float32),
            pltpu.VMEM((SC_BUF_PAD,), jnp.float32),
            pltpu.SemaphoreType.DMA,
            pltpu.SemaphoreType.DMA,
            pltpu.SemaphoreType.DMA,
        ],
        compiler_params=pltpu.CompilerParams(needs_layout_passes=False),
    )(_sc_scores_kernel)
    return kern(w2v, idx3, cidx)


def _tc_loss_kernel(s_ref, o_ref):
    s = s_ref[...]
    col = lax.broadcasted_iota(jnp.int32, s.shape, 1)
    ispos = col < (WINDOW - 1)
    x = jnp.where(ispos, s, -s)
    sg = jax.nn.sigmoid(x)
    sg = jnp.where(ispos, sg, sg + 1e-09 * (sg == 0).astype(jnp.float32))
    l = jnp.log(sg)
    pos_sum = jnp.sum(jnp.where(ispos, l, 0.0))
    neg_sum = jnp.sum(jnp.where(ispos, 0.0, l))
    o_ref[0, 0] = -(pos_sum / (BATCH * (WINDOW - 1))
                    + neg_sum / (BATCH * (WINDOW - 1) * NS))


def _tc_loss(scores):
    out = pl.pallas_call(
        _tc_loss_kernel,
        out_shape=jax.ShapeDtypeStruct((1, 1), jnp.float32),
        out_specs=pl.BlockSpec(memory_space=pltpu.SMEM),
    )(scores)
    return out[0, 0]


def kernel(input, w2v, nsi):
    ctx = jnp.concatenate([input[:TID], input[TID + 1:]], axis=0).T
    neg = jnp.transpose(nsi, (1, 0, 2)).reshape(BATCH, (WINDOW - 1) * NS)
    idx_all = jnp.concatenate([ctx, neg], axis=1).astype(jnp.int32)
    idx3 = idx_all.reshape(NCHUNKS, GATHER_SPLIT, ROWS_PER_GATHER)
    cidx = input[TID].astype(jnp.int32).reshape(NWORKERS, B_PER_W)
    scores = _sc_scores(w2v, idx3, cidx)
    scores = scores[:, :SCORES_PER_W].reshape(BATCH, NSC)
    return _tc_loss(scores)


# speedup vs baseline: 33.3110x; 1.1103x over previous
import functools

import jax
import jax.numpy as jnp
from jax import lax
from jax.experimental import pallas as pl
from jax.experimental.pallas import tpu as pltpu
from jax.experimental.pallas import tpu_sc as plsc

WINDOW = 5
TID = 2
NS = 8
NDIM = 128
BATCH = 4096
NSC = 36
NWORKERS = 32
B_PER_W = BATCH // NWORKERS
CHUNK_B = 8
NCHUNKS = BATCH // CHUNK_B
CHUNKS_PER_W = NCHUNKS // NWORKERS
ROWS_PER_CHUNK = CHUNK_B * NSC
GATHER_SPLIT = 3
ROWS_PER_GATHER = ROWS_PER_CHUNK // GATHER_SPLIT
SCORES_PER_W = B_PER_W * NSC
SC_BUF_PAD = SCORES_PER_W + NDIM
NLANE = 16
NBUF = 2


def _sc_scores_kernel(w2v_hbm, idx3_hbm, cidx_hbm, out_hbm,
                      idx_v, cidx_v, vi_v, rows0_v, rows1_v,
                      sc_v, sem_vi, sem0, sem1):
    wid = lax.axis_index("s") * 2 + lax.axis_index("c")
    rows_bufs = (rows0_v, rows1_v)
    sems = (sem0, sem1)

    pltpu.sync_copy(idx3_hbm.at[pl.ds(wid * CHUNKS_PER_W, CHUNKS_PER_W)], idx_v)
    pltpu.sync_copy(cidx_hbm.at[wid], cidx_v)
    pltpu.async_copy(w2v_hbm.at[cidx_v], vi_v, sem_vi).wait()

    def issue(c, buf, sem):
        for k in range(GATHER_SPLIT):
            pltpu.async_copy(
                w2v_hbm.at[idx_v.at[c].at[k]],
                rows_bufs[buf].at[pl.ds(k * ROWS_PER_GATHER, ROWS_PER_GATHER)],
                sem,
            )

    def drain(buf, sem):
        pltpu.make_async_copy(
            w2v_hbm.at[pl.ds(0, ROWS_PER_CHUNK)], rows_bufs[buf], sem
        ).wait()

    lane0 = lax.iota(jnp.int32, NLANE) == 0

    def _tree_sum(vs):
        while len(vs) > 1:
            vs = [vs[i] + vs[i + 1] for i in range(0, len(vs) - 1, 2)] + (
                [vs[-1]] if len(vs) % 2 else [])
        return vs[0]

    def compute(c, buf):
        rows_v = rows_bufs[buf]

        def store_score(pos, s):
            plsc.store_compressed(
                sc_v.at[pl.ds(pos, NLANE)],
                jnp.full((NLANE,), s, jnp.float32), mask=lane0)

        def b_body(b, carry2):
            row = c * CHUNK_B + b
            vi_regs = [vi_v[row, pl.ds(k * NLANE, NLANE)]
                       for k in range(NDIM // NLANE)]
            base = row * NSC

            def load_rows(j):
                r = b * NSC + j
                return [rows_v[r, pl.ds(k * NLANE, NLANE)]
                        for k in range(NDIM // NLANE)]

            regs = load_rows(0)
            prev_s = None
            for j in range(NSC):
                prods = [regs[k] * vi_regs[k] for k in range(NDIM // NLANE)]
                if j + 1 < NSC:
                    regs = load_rows(j + 1)
                s = jnp.sum(_tree_sum(prods))
                if prev_s is not None:
                    store_score(base + j - 1, prev_s)
                prev_s = s
            store_score(base + NSC - 1, prev_s)
            return carry2

        lax.fori_loop(0, CHUNK_B, b_body, 0, unroll=False)

    for buf in range(NBUF):
        issue(buf, buf, sems[buf])

    def ring_body(it, carry):
        c0 = it * NBUF
        for off in range(NBUF):
            c = c0 + off
            drain(off, sems[off])
            compute(c, off)

            @pl.when(c + NBUF < CHUNKS_PER_W)
            def _():
                issue(c + NBUF, off, sems[off])
        return carry

    lax.fori_loop(0, CHUNKS_PER_W // NBUF, ring_body, 0, unroll=False)
    pltpu.sync_copy(sc_v, out_hbm.at[wid])


def _sc_scores(w2v, idx3, cidx):
    mesh = plsc.VectorSubcoreMesh(core_axis_name="c", subcore_axis_name="s")
    kern = functools.partial(
        pl.kernel,
        mesh=mesh,
        out_type=jax.ShapeDtypeStruct((NWORKERS, SC_BUF_PAD), jnp.float32),
        scratch_types=[
            pltpu.VMEM((CHUNKS_PER_W, GATHER_SPLIT, ROWS_PER_GATHER), jnp.int32),
            pltpu.VMEM((B_PER_W,), jnp.int32),
            pltpu.VMEM((B_PER_W, NDIM), jnp.float32),
            pltpu.VMEM((ROWS_PER_CHUNK, NDIM), jnp.float32),
            pltpu.VMEM((ROWS_PER_CHUNK, NDIM), jnp.float32),
            pltpu.VMEM((SC_BUF_PAD,), jnp.float32),
            pltpu.SemaphoreType.DMA,
            pltpu.SemaphoreType.DMA,
            pltpu.SemaphoreType.DMA,
        ],
        compiler_params=pltpu.CompilerParams(needs_layout_passes=False),
    )(_sc_scores_kernel)
    return kern(w2v, idx3, cidx)


def _tc_loss_kernel(s_ref, o_ref):
    s = s_ref[...]
    col = lax.broadcasted_iota(jnp.int32, s.shape, 1)
    valid = col < SCORES_PER_W
    j = lax.rem(col, NSC)
    ispos = jnp.logical_and(valid, j < (WINDOW - 1))
    isneg = jnp.logical_and(valid, j >= (WINDOW - 1))
    x = jnp.where(ispos, s, -s)
    sg = jax.nn.sigmoid(x)
    sg = jnp.where(ispos, sg, sg + 1e-09 * (sg == 0).astype(jnp.float32))
    l = jnp.log(jnp.where(valid, sg, 1.0))
    pos_sum = jnp.sum(jnp.where(ispos, l, 0.0))
    neg_sum = jnp.sum(jnp.where(isneg, l, 0.0))
    o_ref[0, 0] = -(pos_sum / (BATCH * (WINDOW - 1))
                    + neg_sum / (BATCH * (WINDOW - 1) * NS))


def _tc_loss(scores):
    out = pl.pallas_call(
        _tc_loss_kernel,
        out_shape=jax.ShapeDtypeStruct((1, 1), jnp.float32),
        out_specs=pl.BlockSpec(memory_space=pltpu.SMEM),
    )(scores)
    return out[0, 0]


def kernel(input, w2v, nsi):
    ctx = jnp.concatenate([input[:TID], input[TID + 1:]], axis=0).T
    neg = jnp.transpose(nsi, (1, 0, 2)).reshape(BATCH, (WINDOW - 1) * NS)
    idx_all = jnp.concatenate([ctx, neg], axis=1).astype(jnp.int32)
    idx3 = idx_all.reshape(NCHUNKS, GATHER_SPLIT, ROWS_PER_GATHER)
    cidx = input[TID].astype(jnp.int32).reshape(NWORKERS, B_PER_W)
    scores = _sc_scores(w2v, idx3, cidx)
    return _tc_loss(scores)
